# baseline (speedup 1.0000x reference)
import functools

import jax
import jax.numpy as jnp
from jax import lax
from jax.experimental import pallas as pl
from jax.experimental.pallas import tpu as pltpu
from jax.experimental.pallas import tpu_sc as plsc

E = 8
K = 2
D = 1024
FF = 2048
T = 4096
TK = T * K

NC = 2
NS = 16
NW = NC * NS
LANES = 16

TB = 512
GB = 1024
NBLK = TK // GB
S_STEPS = NBLK + E - 1

TWD = T // NW
CCH = 32


def _route_body(x_ref, gw_ref, logits_ref, selrank_ref, gates_ref, counts_ref,
                carry_ref):
    step = pl.program_id(0)

    @pl.when(step == 0)
    def _():
        carry_ref[...] = jnp.zeros_like(carry_ref)

    xb = x_ref[...]
    gw = gw_ref[...]
    logits = lax.dot_general(xb, gw, (((1,), (1,)), ((), ())),
                             preferred_element_type=jnp.float32)
    logits_ref[...] = logits

    j8 = lax.broadcasted_iota(jnp.int32, (TB, E), 1)
    m0 = jnp.max(logits, axis=1, keepdims=True)
    is0 = logits == m0
    idx0 = jnp.min(jnp.where(is0, j8, E), axis=1, keepdims=True)
    masked = jnp.where(j8 == idx0, -jnp.inf, logits)
    m1 = jnp.max(masked, axis=1, keepdims=True)
    is1 = masked == m1
    idx1 = jnp.min(jnp.where(is1, j8, E), axis=1, keepdims=True)

    ex = jnp.exp(m1 - m0)
    denom = 1.0 + ex
    g0 = 1.0 / denom
    g1 = ex / denom

    oh0 = (j8 == idx0).astype(jnp.float32)
    oh1 = (j8 == idx1).astype(jnp.float32)
    oh = oh0 + oh1

    r_ids = lax.broadcasted_iota(jnp.int32, (TB, TB), 0)
    c_ids = lax.broadcasted_iota(jnp.int32, (TB, TB), 1)
    ltri = (c_ids < r_ids).astype(jnp.float32)
    cum_excl = carry_ref[...] + lax.dot_general(
        ltri, oh, (((1,), (0,)), ((), ())),
        preferred_element_type=jnp.float32)

    rank0 = jnp.sum(oh0 * cum_excl, axis=1, keepdims=True)
    rank1 = jnp.sum(oh1 * cum_excl, axis=1, keepdims=True)

    new_carry = carry_ref[...] + jnp.sum(oh, axis=0, keepdims=True)
    carry_ref[...] = new_carry
    counts_ref[...] = new_carry

    selrank_ref[...] = jnp.concatenate(
        [idx0, idx1, rank0.astype(jnp.int32), rank1.astype(jnp.int32)],
        axis=1)
    gates_ref[...] = jnp.concatenate([g0, g1], axis=1)


def _route(x, gate_w):
    nsteps = T // TB
    return pl.pallas_call(
        _route_body,
        grid=(nsteps,),
        in_specs=[
            pl.BlockSpec((TB, D), lambda s: (s, 0)),
            pl.BlockSpec((E, D), lambda s: (0, 0)),
        ],
        out_specs=[
            pl.BlockSpec((TB, E), lambda s: (s, 0)),
            pl.BlockSpec((TB, 4), lambda s: (s, 0)),
            pl.BlockSpec((TB, 2), lambda s: (s, 0)),
            pl.BlockSpec((1, E), lambda s: (0, 0)),
        ],
        out_shape=[
            jax.ShapeDtypeStruct((T, E), jnp.float32),
            jax.ShapeDtypeStruct((T, 4), jnp.int32),
            jax.ShapeDtypeStruct((T, 2), jnp.float32),
            jax.ShapeDtypeStruct((1, E), jnp.float32),
        ],
        scratch_shapes=[pltpu.VMEM((1, E), jnp.float32)],
    )(x, gate_w)


def _positions(sel0_v, rank0_v, sel1_v, rank1_v, off_v, pos0_v, pos1_v):
    for i in range(TWD // LANES):
        row = (i * LANES) // CCH
        col = (i * LANES) % CCH
        s0 = sel0_v[pl.ds(i * LANES, LANES)]
        r0 = rank0_v[pl.ds(i * LANES, LANES)]
        pos0_v[row, pl.ds(col, LANES)] = r0 + plsc.load_gather(off_v, [s0])
        s1 = sel1_v[pl.ds(i * LANES, LANES)]
        r1 = rank1_v[pl.ds(i * LANES, LANES)]
        pos1_v[row, pl.ds(col, LANES)] = r1 + plsc.load_gather(off_v, [s1])


def _dispatch_body(x_hbm, sel0_hbm, sel1_hbm, rank0_hbm, rank1_hbm, off_hbm,
                   hs_hbm,
                   sel0_v, sel1_v, rank0_v, rank1_v, off_v,
                   pos0_v, pos1_v, xbuf, sem0, sem1):
    cid = lax.axis_index("c")
    sid = lax.axis_index("s")
    wid = sid * NC + cid
    base = wid * TWD

    pltpu.sync_copy(sel0_hbm.at[pl.ds(base, TWD)], sel0_v)
    pltpu.sync_copy(sel1_hbm.at[pl.ds(base, TWD)], sel1_v)
    pltpu.sync_copy(rank0_hbm.at[pl.ds(base, TWD)], rank0_v)
    pltpu.sync_copy(rank1_hbm.at[pl.ds(base, TWD)], rank1_v)
    pltpu.sync_copy(off_hbm, off_v)

    _positions(sel0_v, rank0_v, sel1_v, rank1_v, off_v, pos0_v, pos1_v)

    for ci in range(TWD // CCH):
        pltpu.sync_copy(x_hbm.at[pl.ds(base + ci * CCH, CCH)], xbuf)
        cp0 = pltpu.make_async_copy(xbuf, hs_hbm.at[pos0_v.at[ci]], sem0)
        cp1 = pltpu.make_async_copy(xbuf, hs_hbm.at[pos1_v.at[ci]], sem1)
        cp0.start()
        cp1.start()
        cp0.wait()
        cp1.wait()


def _dispatch(x, sel0, sel1, rank0, rank1, off16):
    mesh = plsc.VectorSubcoreMesh(core_axis_name="c", subcore_axis_name="s")
    kern = pl.kernel(
        _dispatch_body,
        out_type=jax.ShapeDtypeStruct((TK, D), jnp.float32),
        mesh=mesh,
        compiler_params=pltpu.CompilerParams(needs_layout_passes=False),
        scratch_types=[
            pltpu.VMEM((TWD,), jnp.int32),
            pltpu.VMEM((TWD,), jnp.int32),
            pltpu.VMEM((TWD,), jnp.int32),
            pltpu.VMEM((TWD,), jnp.int32),
            pltpu.VMEM((16,), jnp.int32),
            pltpu.VMEM((TWD // CCH, CCH), jnp.int32),
            pltpu.VMEM((TWD // CCH, CCH), jnp.int32),
            pltpu.VMEM((CCH, D), jnp.float32),
            pltpu.SemaphoreType.DMA,
            pltpu.SemaphoreType.DMA,
        ],
    )
    return kern(x, sel0, sel1, rank0, rank1, off16)


def _erf(x):
    return lax.erf(x)


def _gelu(x):
    return 0.5 * x * (1.0 + _erf(x * 0.7071067811865476))


def _gemm_body(meta_ref, h_ref, wfc_ref, bfc_ref, wproj_ref, bproj_ref,
               out_ref):
    s = pl.program_id(0)
    lo = meta_ref[2, s]
    hi = meta_ref[3, s]
    rb = meta_ref[1, s]
    rb_prev = meta_ref[1, jnp.maximum(s - 1, 0)]
    first = jnp.logical_or(s == 0, rb != rb_prev)

    hb = h_ref[...]
    he = lax.dot_general(hb, wfc_ref[0], (((1,), (1,)), ((), ())),
                         preferred_element_type=jnp.float32)
    he = he + bfc_ref[0]
    he = _gelu(he)
    yb = lax.dot_general(he, wproj_ref[0], (((1,), (1,)), ((), ())),
                         preferred_element_type=jnp.float32)
    yb = yb + bproj_ref[0]

    rows = lax.broadcasted_iota(jnp.int32, (GB, 1), 0)
    mask = jnp.logical_and(rows >= lo, rows < hi)

    @pl.when(first)
    def _():
        out_ref[...] = jnp.where(mask, yb, 0.0)

    @pl.when(jnp.logical_not(first))
    def _():
        out_ref[...] = jnp.where(mask, yb, out_ref[...])


def _grouped_gemm(meta, h_sorted, w_fc, b_fc, w_proj, b_proj):
    grid_spec = pltpu.PrefetchScalarGridSpec(
        num_scalar_prefetch=1,
        grid=(S_STEPS,),
        in_specs=[
            pl.BlockSpec((GB, D), lambda s, m: (m[1, s], 0)),
            pl.BlockSpec((1, FF, D), lambda s, m: (m[0, s], 0, 0)),
            pl.BlockSpec((1, 1, FF), lambda s, m: (m[0, s], 0, 0)),
            pl.BlockSpec((1, D, FF), lambda s, m: (m[0, s], 0, 0)),
            pl.BlockSpec((1, 1, D), lambda s, m: (m[0, s], 0, 0)),
        ],
        out_specs=pl.BlockSpec((GB, D), lambda s, m: (m[1, s], 0)),
    )
    return pl.pallas_call(
        _gemm_body,
        grid_spec=grid_spec,
        out_shape=jax.ShapeDtypeStruct((TK, D), jnp.float32),
    )(meta, h_sorted, w_fc, b_fc.reshape(E, 1, FF), w_proj,
      b_proj.reshape(E, 1, D))


def _combine_body(y_hbm, sel0_hbm, sel1_hbm, rank0_hbm, rank1_hbm, off_hbm,
                  g0_hbm, g1_hbm,
                  out_hbm,
                  sel0_v, sel1_v, rank0_v, rank1_v, off_v,
                  pos0_v, pos1_v, g0_v, g1_v,
                  buf0, buf1, obuf, sem0, sem1):
    cid = lax.axis_index("c")
    sid = lax.axis_index("s")
    wid = sid * NC + cid
    base = wid * TWD

    pltpu.sync_copy(sel0_hbm.at[pl.ds(base, TWD)], sel0_v)
    pltpu.sync_copy(sel1_hbm.at[pl.ds(base, TWD)], sel1_v)
    pltpu.sync_copy(rank0_hbm.at[pl.ds(base, TWD)], rank0_v)
    pltpu.sync_copy(rank1_hbm.at[pl.ds(base, TWD)], rank1_v)
    pltpu.sync_copy(g0_hbm.at[pl.ds(base, TWD)], g0_v)
    pltpu.sync_copy(g1_hbm.at[pl.ds(base, TWD)], g1_v)
    pltpu.sync_copy(off_hbm, off_v)

    _positions(sel0_v, rank0_v, sel1_v, rank1_v, off_v, pos0_v, pos1_v)

    for ci in range(TWD // CCH):
        cp0 = pltpu.make_async_copy(y_hbm.at[pos0_v.at[ci]], buf0, sem0)
        cp1 = pltpu.make_async_copy(y_hbm.at[pos1_v.at[ci]], buf1, sem1)
        cp0.start()
        cp1.start()
        cp0.wait()
        cp1.wait()

        def row_body(r, _):
            idx16 = jnp.zeros((LANES,), jnp.int32) + (ci * CCH + r)
            g0v = plsc.load_gather(g0_v, [idx16])
            g1v = plsc.load_gather(g1_v, [idx16])
            for j in range(D // LANES):
                a = buf0[r, pl.ds(j * LANES, LANES)]
                b = buf1[r, pl.ds(j * LANES, LANES)]
                obuf[r, pl.ds(j * LANES, LANES)] = a * g0v + b * g1v
            return 0

        lax.fori_loop(0, CCH, row_body, 0)
        pltpu.sync_copy(obuf, out_hbm.at[pl.ds(base + ci * CCH, CCH)])


def _combine(y_sorted, sel0, sel1, rank0, rank1, off16, g0, g1):
    mesh = plsc.VectorSubcoreMesh(core_axis_name="c", subcore_axis_name="s")
    kern = pl.kernel(
        _combine_body,
        out_type=jax.ShapeDtypeStruct((T, D), jnp.float32),
        mesh=mesh,
        compiler_params=pltpu.CompilerParams(needs_layout_passes=False),
        scratch_types=[
            pltpu.VMEM((TWD,), jnp.int32),
            pltpu.VMEM((TWD,), jnp.int32),
            pltpu.VMEM((TWD,), jnp.int32),
            pltpu.VMEM((TWD,), jnp.int32),
            pltpu.VMEM((16,), jnp.int32),
            pltpu.VMEM((TWD // CCH, CCH), jnp.int32),
            pltpu.VMEM((TWD // CCH, CCH), jnp.int32),
            pltpu.VMEM((TWD,), jnp.float32),
            pltpu.VMEM((TWD,), jnp.float32),
            pltpu.VMEM((CCH, D), jnp.float32),
            pltpu.VMEM((CCH, D), jnp.float32),
            pltpu.VMEM((CCH, D), jnp.float32),
            pltpu.SemaphoreType.DMA,
            pltpu.SemaphoreType.DMA,
        ],
    )
    return kern(y_sorted, sel0, sel1, rank0, rank1, off16, g0, g1)


def _gemm_meta(counts):
    counts = counts.astype(jnp.int32)
    off = jnp.concatenate([jnp.zeros((1,), jnp.int32), jnp.cumsum(counts)])
    has = counts > 0
    bstart = off[:E] // GB
    bend = (off[1:] - 1) // GB
    nsteps = jnp.where(has, bend - bstart + 1, 0)
    first = jnp.cumsum(nsteps) - nsteps
    cum_end = first + nsteps
    total = cum_end[E - 1]
    s_ids = jnp.arange(S_STEPS, dtype=jnp.int32)
    e_of = jnp.sum((s_ids[:, None] >= cum_end[None, :]).astype(jnp.int32),
                   axis=1)
    e_s = jnp.clip(e_of, 0, E - 1)
    b_s = jnp.take(bstart, e_s) + (s_ids - jnp.take(first, e_s))
    lo = jnp.maximum(jnp.take(off, e_s) - b_s * GB, 0)
    hi = jnp.minimum(jnp.take(off, e_s + 1) - b_s * GB, GB)
    valid = s_ids < total
    e_last = jnp.sum(jnp.where(s_ids == total - 1, e_s, 0))
    b_last = jnp.sum(jnp.where(s_ids == total - 1, b_s, 0))
    e_s = jnp.where(valid, e_s, e_last)
    b_s = jnp.where(valid, b_s, b_last)
    lo = jnp.where(valid, lo, 0)
    hi = jnp.where(valid, hi, 0)
    return jnp.stack([e_s, b_s, lo, hi]).astype(jnp.int32), off


def kernel(hidden_states, gate_w, w_fc, b_fc, w_proj, b_proj):
    orig_shape = hidden_states.shape
    x = hidden_states.reshape(-1, D)

    logits, selrank, gates, counts_f = _route(x, gate_w)

    sel0 = selrank[:, 0]
    sel1 = selrank[:, 1]
    rank0 = selrank[:, 2]
    rank1 = selrank[:, 3]
    g0 = gates[:, 0]
    g1 = gates[:, 1]

    meta, off = _gemm_meta(counts_f[0])
    off16 = jnp.pad(off[:E], (0, 16 - E)).astype(jnp.int32)

    h_sorted = _dispatch(x, sel0, sel1, rank0, rank1, off16)
    y_sorted = _grouped_gemm(meta, h_sorted, w_fc, b_fc, w_proj, b_proj)
    out = _combine(y_sorted, sel0, sel1, rank0, rank1, off16, g0, g1)

    return (out.reshape(orig_shape), logits)


# speedup vs baseline: 1.1200x; 1.1200x over previous
import functools

import jax
import jax.numpy as jnp
from jax import lax
from jax.experimental import pallas as pl
from jax.experimental.pallas import tpu as pltpu
from jax.experimental.pallas import tpu_sc as plsc

E = 8
K = 2
D = 1024
FF = 2048
T = 4096
TK = T * K

NC = 2
NS = 16
NW = NC * NS
LANES = 16

TB = 512
GB = 512
NBLK = TK // GB
S_STEPS = NBLK + E - 1

TWD = T // NW
CCH = 16


def _route_body(x_ref, gw_ref, logits_ref, selrank_ref, gates_ref, counts_ref,
                carry_ref):
    step = pl.program_id(0)

    @pl.when(step == 0)
    def _():
        carry_ref[...] = jnp.zeros_like(carry_ref)

    xb = x_ref[...]
    gw = gw_ref[...]
    logits = lax.dot_general(xb, gw, (((1,), (1,)), ((), ())),
                             preferred_element_type=jnp.float32)
    logits_ref[...] = logits

    j8 = lax.broadcasted_iota(jnp.int32, (TB, E), 1)
    m0 = jnp.max(logits, axis=1, keepdims=True)
    is0 = logits == m0
    idx0 = jnp.min(jnp.where(is0, j8, E), axis=1, keepdims=True)
    masked = jnp.where(j8 == idx0, -jnp.inf, logits)
    m1 = jnp.max(masked, axis=1, keepdims=True)
    is1 = masked == m1
    idx1 = jnp.min(jnp.where(is1, j8, E), axis=1, keepdims=True)

    ex = jnp.exp(m1 - m0)
    denom = 1.0 + ex
    g0 = 1.0 / denom
    g1 = ex / denom

    oh0 = (j8 == idx0).astype(jnp.float32)
    oh1 = (j8 == idx1).astype(jnp.float32)
    oh = oh0 + oh1

    r_ids = lax.broadcasted_iota(jnp.int32, (TB, TB), 0)
    c_ids = lax.broadcasted_iota(jnp.int32, (TB, TB), 1)
    ltri = (c_ids < r_ids).astype(jnp.float32)
    cum_excl = carry_ref[...] + lax.dot_general(
        ltri, oh, (((1,), (0,)), ((), ())),
        preferred_element_type=jnp.float32)

    rank0 = jnp.sum(oh0 * cum_excl, axis=1, keepdims=True)
    rank1 = jnp.sum(oh1 * cum_excl, axis=1, keepdims=True)

    new_carry = carry_ref[...] + jnp.sum(oh, axis=0, keepdims=True)
    carry_ref[...] = new_carry
    counts_ref[...] = new_carry

    selrank_ref[...] = jnp.concatenate(
        [idx0, idx1, rank0.astype(jnp.int32), rank1.astype(jnp.int32)],
        axis=1)
    gates_ref[...] = jnp.concatenate([g0, g1], axis=1)


def _route(x, gate_w):
    nsteps = T // TB
    return pl.pallas_call(
        _route_body,
        grid=(nsteps,),
        in_specs=[
            pl.BlockSpec((TB, D), lambda s: (s, 0)),
            pl.BlockSpec((E, D), lambda s: (0, 0)),
        ],
        out_specs=[
            pl.BlockSpec((TB, E), lambda s: (s, 0)),
            pl.BlockSpec((TB, 4), lambda s: (s, 0)),
            pl.BlockSpec((TB, 2), lambda s: (s, 0)),
            pl.BlockSpec((1, E), lambda s: (0, 0)),
        ],
        out_shape=[
            jax.ShapeDtypeStruct((T, E), jnp.float32),
            jax.ShapeDtypeStruct((T, 4), jnp.int32),
            jax.ShapeDtypeStruct((T, 2), jnp.float32),
            jax.ShapeDtypeStruct((1, E), jnp.float32),
        ],
        scratch_shapes=[pltpu.VMEM((1, E), jnp.float32)],
    )(x, gate_w)


def _positions(sel0_v, rank0_v, sel1_v, rank1_v, off_v, pos0_v, pos1_v):
    for i in range(TWD // LANES):
        row = (i * LANES) // CCH
        col = (i * LANES) % CCH
        s0 = sel0_v[pl.ds(i * LANES, LANES)]
        r0 = rank0_v[pl.ds(i * LANES, LANES)]
        pos0_v[row, pl.ds(col, LANES)] = r0 + plsc.load_gather(off_v, [s0])
        s1 = sel1_v[pl.ds(i * LANES, LANES)]
        r1 = rank1_v[pl.ds(i * LANES, LANES)]
        pos1_v[row, pl.ds(col, LANES)] = r1 + plsc.load_gather(off_v, [s1])


def _dispatch_body(x_hbm, sel0_hbm, sel1_hbm, rank0_hbm, rank1_hbm, off_hbm,
                   hs_hbm,
                   sel0_v, sel1_v, rank0_v, rank1_v, off_v,
                   pos0_v, pos1_v, xbuf, seml, sems):
    cid = lax.axis_index("c")
    sid = lax.axis_index("s")
    wid = sid * NC + cid
    base = wid * TWD

    pltpu.sync_copy(sel0_hbm.at[pl.ds(base, TWD)], sel0_v)
    pltpu.sync_copy(sel1_hbm.at[pl.ds(base, TWD)], sel1_v)
    pltpu.sync_copy(rank0_hbm.at[pl.ds(base, TWD)], rank0_v)
    pltpu.sync_copy(rank1_hbm.at[pl.ds(base, TWD)], rank1_v)
    pltpu.sync_copy(off_hbm, off_v)

    _positions(sel0_v, rank0_v, sel1_v, rank1_v, off_v, pos0_v, pos1_v)

    nch = TWD // CCH

    def load(ci):
        b = ci % 2
        return pltpu.make_async_copy(
            x_hbm.at[pl.ds(base + ci * CCH, CCH)], xbuf.at[b], seml.at[b])

    def scat(ci, slot):
        b = ci % 2
        pos = pos0_v if slot == 0 else pos1_v
        return pltpu.make_async_copy(
            xbuf.at[b], hs_hbm.at[pos.at[ci]], sems.at[b])

    load(0).start()
    for ci in range(nch):
        load(ci).wait()
        if ci + 1 < nch:
            if ci >= 1:
                scat(ci - 1, 0).wait()
                scat(ci - 1, 1).wait()
            load(ci + 1).start()
        scat(ci, 0).start()
        scat(ci, 1).start()
    if nch >= 2:
        scat(nch - 2, 0).wait()
        scat(nch - 2, 1).wait()
    scat(nch - 1, 0).wait()
    scat(nch - 1, 1).wait()


def _dispatch(x, sel0, sel1, rank0, rank1, off16):
    mesh = plsc.VectorSubcoreMesh(core_axis_name="c", subcore_axis_name="s")
    kern = pl.kernel(
        _dispatch_body,
        out_type=jax.ShapeDtypeStruct((TK, D), jnp.float32),
        mesh=mesh,
        compiler_params=pltpu.CompilerParams(needs_layout_passes=False),
        scratch_types=[
            pltpu.VMEM((TWD,), jnp.int32),
            pltpu.VMEM((TWD,), jnp.int32),
            pltpu.VMEM((TWD,), jnp.int32),
            pltpu.VMEM((TWD,), jnp.int32),
            pltpu.VMEM((16,), jnp.int32),
            pltpu.VMEM((TWD // CCH, CCH), jnp.int32),
            pltpu.VMEM((TWD // CCH, CCH), jnp.int32),
            pltpu.VMEM((2, CCH, D), jnp.float32),
            pltpu.SemaphoreType.DMA((2,)),
            pltpu.SemaphoreType.DMA((2,)),
        ],
    )
    return kern(x, sel0, sel1, rank0, rank1, off16)


def _erf(x):
    return lax.erf(x)


def _gelu(x):
    return 0.5 * x * (1.0 + _erf(x * 0.7071067811865476))


def _gemm_body(meta_ref, h_ref, wfc_ref, bfc_ref, wproj_ref, bproj_ref,
               out_ref):
    s = pl.program_id(0)
    lo = meta_ref[2, s]
    hi = meta_ref[3, s]
    rb = meta_ref[1, s]
    rb_prev = meta_ref[1, jnp.maximum(s - 1, 0)]
    first = jnp.logical_or(s == 0, rb != rb_prev)

    hb = h_ref[...]
    he = lax.dot_general(hb, wfc_ref[0], (((1,), (1,)), ((), ())),
                         preferred_element_type=jnp.float32)
    he = he + bfc_ref[0]
    he = _gelu(he)
    yb = lax.dot_general(he, wproj_ref[0], (((1,), (1,)), ((), ())),
                         preferred_element_type=jnp.float32)
    yb = yb + bproj_ref[0]

    rows = lax.broadcasted_iota(jnp.int32, (GB, 1), 0)
    mask = jnp.logical_and(rows >= lo, rows < hi)

    @pl.when(first)
    def _():
        out_ref[...] = jnp.where(mask, yb, 0.0)

    @pl.when(jnp.logical_not(first))
    def _():
        out_ref[...] = jnp.where(mask, yb, out_ref[...])


def _grouped_gemm(meta, h_sorted, w_fc, b_fc, w_proj, b_proj):
    grid_spec = pltpu.PrefetchScalarGridSpec(
        num_scalar_prefetch=1,
        grid=(S_STEPS,),
        in_specs=[
            pl.BlockSpec((GB, D), lambda s, m: (m[1, s], 0)),
            pl.BlockSpec((1, FF, D), lambda s, m: (m[0, s], 0, 0)),
            pl.BlockSpec((1, 1, FF), lambda s, m: (m[0, s], 0, 0)),
            pl.BlockSpec((1, D, FF), lambda s, m: (m[0, s], 0, 0)),
            pl.BlockSpec((1, 1, D), lambda s, m: (m[0, s], 0, 0)),
        ],
        out_specs=pl.BlockSpec((GB, D), lambda s, m: (m[1, s], 0)),
    )
    return pl.pallas_call(
        _gemm_body,
        grid_spec=grid_spec,
        out_shape=jax.ShapeDtypeStruct((TK, D), jnp.float32),
    )(meta, h_sorted, w_fc, b_fc.reshape(E, 1, FF), w_proj,
      b_proj.reshape(E, 1, D))


def _combine_body(y_hbm, sel0_hbm, sel1_hbm, rank0_hbm, rank1_hbm, off_hbm,
                  g0_hbm, g1_hbm,
                  out_hbm,
                  sel0_v, sel1_v, rank0_v, rank1_v, off_v,
                  pos0_v, pos1_v, g0_v, g1_v,
                  buf0, buf1, obuf, semg, semo):
    cid = lax.axis_index("c")
    sid = lax.axis_index("s")
    wid = sid * NC + cid
    base = wid * TWD

    pltpu.sync_copy(sel0_hbm.at[pl.ds(base, TWD)], sel0_v)
    pltpu.sync_copy(sel1_hbm.at[pl.ds(base, TWD)], sel1_v)
    pltpu.sync_copy(rank0_hbm.at[pl.ds(base, TWD)], rank0_v)
    pltpu.sync_copy(rank1_hbm.at[pl.ds(base, TWD)], rank1_v)
    pltpu.sync_copy(g0_hbm.at[pl.ds(base, TWD)], g0_v)
    pltpu.sync_copy(g1_hbm.at[pl.ds(base, TWD)], g1_v)
    pltpu.sync_copy(off_hbm, off_v)

    _positions(sel0_v, rank0_v, sel1_v, rank1_v, off_v, pos0_v, pos1_v)

    nch = TWD // CCH

    def gather(ci):
        b = ci % 2
        c0 = pltpu.make_async_copy(y_hbm.at[pos0_v.at[ci]], buf0.at[b],
                                   semg.at[b])
        c1 = pltpu.make_async_copy(y_hbm.at[pos1_v.at[ci]], buf1.at[b],
                                   semg.at[b])
        return c0, c1

    def store(ci):
        b = ci % 2
        return pltpu.make_async_copy(
            obuf.at[b], out_hbm.at[pl.ds(base + ci * CCH, CCH)], semo.at[b])

    g0c, g1c = gather(0)
    g0c.start()
    g1c.start()
    for ci in range(nch):
        b = ci % 2
        c0, c1 = gather(ci)
        c0.wait()
        c1.wait()
        if ci + 1 < nch:
            n0, n1 = gather(ci + 1)
            n0.start()
            n1.start()
        if ci >= 2:
            store(ci - 2).wait()

        def row_body(r, _):
            idx16 = jnp.zeros((LANES,), jnp.int32) + (ci * CCH + r)
            g0v = plsc.load_gather(g0_v, [idx16])
            g1v = plsc.load_gather(g1_v, [idx16])
            for j in range(D // LANES):
                a = buf0[b, r, pl.ds(j * LANES, LANES)]
                bb = buf1[b, r, pl.ds(j * LANES, LANES)]
                obuf[b, r, pl.ds(j * LANES, LANES)] = a * g0v + bb * g1v
            return 0

        lax.fori_loop(0, CCH, row_body, 0)
        store(ci).start()
    if nch >= 2:
        store(nch - 2).wait()
    store(nch - 1).wait()


def _combine(y_sorted, sel0, sel1, rank0, rank1, off16, g0, g1):
    mesh = plsc.VectorSubcoreMesh(core_axis_name="c", subcore_axis_name="s")
    kern = pl.kernel(
        _combine_body,
        out_type=jax.ShapeDtypeStruct((T, D), jnp.float32),
        mesh=mesh,
        compiler_params=pltpu.CompilerParams(needs_layout_passes=False),
        scratch_types=[
            pltpu.VMEM((TWD,), jnp.int32),
            pltpu.VMEM((TWD,), jnp.int32),
            pltpu.VMEM((TWD,), jnp.int32),
            pltpu.VMEM((TWD,), jnp.int32),
            pltpu.VMEM((16,), jnp.int32),
            pltpu.VMEM((TWD // CCH, CCH), jnp.int32),
            pltpu.VMEM((TWD // CCH, CCH), jnp.int32),
            pltpu.VMEM((TWD,), jnp.float32),
            pltpu.VMEM((TWD,), jnp.float32),
            pltpu.VMEM((2, CCH, D), jnp.float32),
            pltpu.VMEM((2, CCH, D), jnp.float32),
            pltpu.VMEM((2, CCH, D), jnp.float32),
            pltpu.SemaphoreType.DMA((2,)),
            pltpu.SemaphoreType.DMA((2,)),
        ],
    )
    return kern(y_sorted, sel0, sel1, rank0, rank1, off16, g0, g1)


def _gemm_meta(counts):
    counts = counts.astype(jnp.int32)
    off = jnp.concatenate([jnp.zeros((1,), jnp.int32), jnp.cumsum(counts)])
    has = counts > 0
    bstart = off[:E] // GB
    bend = (off[1:] - 1) // GB
    nsteps = jnp.where(has, bend - bstart + 1, 0)
    first = jnp.cumsum(nsteps) - nsteps
    cum_end = first + nsteps
    total = cum_end[E - 1]
    s_ids = jnp.arange(S_STEPS, dtype=jnp.int32)
    e_of = jnp.sum((s_ids[:, None] >= cum_end[None, :]).astype(jnp.int32),
                   axis=1)
    e_s = jnp.clip(e_of, 0, E - 1)
    b_s = jnp.take(bstart, e_s) + (s_ids - jnp.take(first, e_s))
    lo = jnp.maximum(jnp.take(off, e_s) - b_s * GB, 0)
    hi = jnp.minimum(jnp.take(off, e_s + 1) - b_s * GB, GB)
    valid = s_ids < total
    e_last = jnp.sum(jnp.where(s_ids == total - 1, e_s, 0))
    b_last = jnp.sum(jnp.where(s_ids == total - 1, b_s, 0))
    e_s = jnp.where(valid, e_s, e_last)
    b_s = jnp.where(valid, b_s, b_last)
    lo = jnp.where(valid, lo, 0)
    hi = jnp.where(valid, hi, 0)
    return jnp.stack([e_s, b_s, lo, hi]).astype(jnp.int32), off


def kernel(hidden_states, gate_w, w_fc, b_fc, w_proj, b_proj):
    orig_shape = hidden_states.shape
    x = hidden_states.reshape(-1, D)

    logits, selrank, gates, counts_f = _route(x, gate_w)

    sel0 = selrank[:, 0]
    sel1 = selrank[:, 1]
    rank0 = selrank[:, 2]
    rank1 = selrank[:, 3]
    g0 = gates[:, 0]
    g1 = gates[:, 1]

    meta, off = _gemm_meta(counts_f[0])
    off16 = jnp.pad(off[:E], (0, 16 - E)).astype(jnp.int32)

    h_sorted = _dispatch(x, sel0, sel1, rank0, rank1, off16)
    y_sorted = _grouped_gemm(meta, h_sorted, w_fc, b_fc, w_proj, b_proj)
    out = _combine(y_sorted, sel0, sel1, rank0, rank1, off16, g0, g1)

    return (out.reshape(orig_shape), logits)


# speedup vs baseline: 1.1613x; 1.0368x over previous
import jax
import jax.numpy as jnp
from jax import lax
from jax.experimental import pallas as pl
from jax.experimental.pallas import tpu as pltpu
from jax.experimental.pallas import tpu_sc as plsc

E = 8
K = 2
D = 1024
FF = 2048
T = 4096
TK = T * K

NC = 2
NS = 16
NW = NC * NS
LANES = 16

TB = 512
GB = 512
NBLK = TK // GB
S_STEPS = NBLK + E - 1

TWD = T // NW
CCH = 16


def _route_body(x_ref, gw_ref, logits_ref, selrank_ref, gates_ref, meta_ref,
                offp_ref, carry_ref):
    step = pl.program_id(0)
    nsteps = pl.num_programs(0)

    @pl.when(step == 0)
    def _():
        carry_ref[...] = jnp.zeros_like(carry_ref)

    xb = x_ref[...]
    gw = gw_ref[...]
    logits = lax.dot_general(xb, gw, (((1,), (1,)), ((), ())),
                             preferred_element_type=jnp.float32)
    logits_ref[...] = logits

    j8 = lax.broadcasted_iota(jnp.int32, (TB, E), 1)
    m0 = jnp.max(logits, axis=1, keepdims=True)
    is0 = logits == m0
    idx0 = jnp.min(jnp.where(is0, j8, E), axis=1, keepdims=True)
    masked = jnp.where(j8 == idx0, -jnp.inf, logits)
    m1 = jnp.max(masked, axis=1, keepdims=True)
    is1 = masked == m1
    idx1 = jnp.min(jnp.where(is1, j8, E), axis=1, keepdims=True)

    ex = jnp.exp(m1 - m0)
    denom = 1.0 + ex
    g0 = 1.0 / denom
    g1 = ex / denom

    oh0 = (j8 == idx0).astype(jnp.float32)
    oh1 = (j8 == idx1).astype(jnp.float32)
    oh = oh0 + oh1

    r_ids = lax.broadcasted_iota(jnp.int32, (TB, TB), 0)
    c_ids = lax.broadcasted_iota(jnp.int32, (TB, TB), 1)
    ltri = (c_ids < r_ids).astype(jnp.float32)
    cum_excl = carry_ref[...] + lax.dot_general(
        ltri, oh, (((1,), (0,)), ((), ())),
        preferred_element_type=jnp.float32)

    rank0 = jnp.sum(oh0 * cum_excl, axis=1, keepdims=True)
    rank1 = jnp.sum(oh1 * cum_excl, axis=1, keepdims=True)

    new_carry = carry_ref[...] + jnp.sum(oh, axis=0, keepdims=True)
    carry_ref[...] = new_carry

    selrank_ref[...] = jnp.concatenate(
        [idx0, idx1, rank0.astype(jnp.int32), rank1.astype(jnp.int32)],
        axis=1)
    gates_ref[...] = jnp.concatenate([g0, g1], axis=1)

    @pl.when(step == nsteps - 1)
    def _():
        counts_f = new_carry
        ka = lax.broadcasted_iota(jnp.int32, (E, E), 0)
        eb = lax.broadcasted_iota(jnp.int32, (E, E), 1)
        mtri = (ka < eb).astype(jnp.float32)
        off_f = lax.dot_general(counts_f, mtri, (((1,), (0,)), ((), ())),
                                preferred_element_type=jnp.float32)
        counts_i = counts_f.astype(jnp.int32)
        off_i = off_f.astype(jnp.int32)
        off1_i = off_i + counts_i
        bstart = off_i // GB
        bend = (off1_i - 1) // GB
        nst = jnp.where(counts_i > 0, bend - bstart + 1, 0)
        first_f = lax.dot_general(nst.astype(jnp.float32), mtri,
                                  (((1,), (0,)), ((), ())),
                                  preferred_element_type=jnp.float32)
        first = first_f.astype(jnp.int32)
        cum_end = first + nst
        total = cum_end[:, E - 1:E]

        s2 = lax.broadcasted_iota(jnp.int32, (S_STEPS, E), 0)
        e2 = lax.broadcasted_iota(jnp.int32, (S_STEPS, E), 1)
        ge = (s2 >= cum_end).astype(jnp.int32)
        e_of = jnp.sum(ge, axis=1, keepdims=True)
        e_s = jnp.clip(e_of, 0, E - 1)
        onehot = (e2 == e_s).astype(jnp.float32)

        def pick(v_i):
            return jnp.sum(onehot * v_i.astype(jnp.float32), axis=1,
                           keepdims=True).astype(jnp.int32)

        s_col = lax.broadcasted_iota(jnp.int32, (S_STEPS, 1), 0)
        b_s = pick(bstart) + s_col - pick(first)
        lo = jnp.maximum(pick(off_i) - b_s * GB, 0)
        hi = jnp.minimum(pick(off1_i) - b_s * GB, GB)
        valid = s_col < total
        at_last = (s_col == total - 1)
        e_last = jnp.sum(jnp.where(at_last, e_s, 0), axis=0, keepdims=True)
        b_last = jnp.sum(jnp.where(at_last, b_s, 0), axis=0, keepdims=True)
        meta_ref[...] = jnp.concatenate(
            [jnp.where(valid, e_s, e_last),
             jnp.where(valid, b_s, b_last),
             jnp.where(valid, lo, 0),
             jnp.where(valid, hi, 0)], axis=1)
        offp_ref[...] = jnp.concatenate(
            [off_i, jnp.zeros((1, 16 - E), jnp.int32)], axis=1)


def _route(x, gate_w):
    nsteps = T // TB
    return pl.pallas_call(
        _route_body,
        grid=(nsteps,),
        in_specs=[
            pl.BlockSpec((TB, D), lambda s: (s, 0)),
            pl.BlockSpec((E, D), lambda s: (0, 0)),
        ],
        out_specs=[
            pl.BlockSpec((TB, E), lambda s: (s, 0)),
            pl.BlockSpec((TB, 4), lambda s: (s, 0)),
            pl.BlockSpec((TB, 2), lambda s: (s, 0)),
            pl.BlockSpec((S_STEPS, 4), lambda s: (0, 0)),
            pl.BlockSpec((1, 16), lambda s: (0, 0)),
        ],
        out_shape=[
            jax.ShapeDtypeStruct((T, E), jnp.float32),
            jax.ShapeDtypeStruct((T, 4), jnp.int32),
            jax.ShapeDtypeStruct((T, 2), jnp.float32),
            jax.ShapeDtypeStruct((S_STEPS, 4), jnp.int32),
            jax.ShapeDtypeStruct((1, 16), jnp.int32),
        ],
        scratch_shapes=[pltpu.VMEM((1, E), jnp.float32)],
    )(x, gate_w)


def _positions(sr_v, off_v, pos0_v, pos1_v):
    lane = lax.iota(jnp.int32, LANES)
    zero = jnp.zeros((LANES,), jnp.int32)
    for i in range(TWD // LANES):
        ti = lane + i * LANES
        s0 = plsc.load_gather(sr_v, [ti, zero])
        s1 = plsc.load_gather(sr_v, [ti, zero + 1])
        r0 = plsc.load_gather(sr_v, [ti, zero + 2])
        r1 = plsc.load_gather(sr_v, [ti, zero + 3])
        row = (i * LANES) // CCH
        col = (i * LANES) % CCH
        pos0_v[row, pl.ds(col, LANES)] = r0 + plsc.load_gather(off_v, [s0])
        pos1_v[row, pl.ds(col, LANES)] = r1 + plsc.load_gather(off_v, [s1])


def _dispatch_body(x_hbm, sr_hbm, offp_hbm, hs_hbm,
                   sr_v, off_v, pos0_v, pos1_v, xbuf, seml, sems):
    cid = lax.axis_index("c")
    sid = lax.axis_index("s")
    wid = sid * NC + cid
    base = wid * TWD

    pltpu.sync_copy(sr_hbm.at[pl.ds(base, TWD)], sr_v)
    pltpu.sync_copy(offp_hbm.at[0], off_v)

    _positions(sr_v, off_v, pos0_v, pos1_v)

    nch = TWD // CCH

    def load(ci):
        b = ci % 2
        return pltpu.make_async_copy(
            x_hbm.at[pl.ds(base + ci * CCH, CCH)], xbuf.at[b], seml.at[b])

    def scat(ci, slot):
        b = ci % 2
        pos = pos0_v if slot == 0 else pos1_v
        return pltpu.make_async_copy(
            xbuf.at[b], hs_hbm.at[pos.at[ci]], sems.at[b])

    load(0).start()
    for ci in range(nch):
        load(ci).wait()
        if ci + 1 < nch:
            if ci >= 1:
                scat(ci - 1, 0).wait()
                scat(ci - 1, 1).wait()
            load(ci + 1).start()
        scat(ci, 0).start()
        scat(ci, 1).start()
    if nch >= 2:
        scat(nch - 2, 0).wait()
        scat(nch - 2, 1).wait()
    scat(nch - 1, 0).wait()
    scat(nch - 1, 1).wait()


def _dispatch(x, selrank, offp):
    mesh = plsc.VectorSubcoreMesh(core_axis_name="c", subcore_axis_name="s")
    kern = pl.kernel(
        _dispatch_body,
        out_type=jax.ShapeDtypeStruct((TK, D), jnp.float32),
        mesh=mesh,
        compiler_params=pltpu.CompilerParams(needs_layout_passes=False),
        scratch_types=[
            pltpu.VMEM((TWD, 4), jnp.int32),
            pltpu.VMEM((16,), jnp.int32),
            pltpu.VMEM((TWD // CCH, CCH), jnp.int32),
            pltpu.VMEM((TWD // CCH, CCH), jnp.int32),
            pltpu.VMEM((2, CCH, D), jnp.float32),
            pltpu.SemaphoreType.DMA((2,)),
            pltpu.SemaphoreType.DMA((2,)),
        ],
    )
    return kern(x, selrank, offp)


def _gelu(x):
    return 0.5 * x * (1.0 + lax.erf(x * 0.7071067811865476))


def _gemm_body(meta_ref, h_ref, wfc_ref, bfc_ref, wproj_ref, bproj_ref,
               out_ref):
    s = pl.program_id(0)
    lo = meta_ref[s, 2]
    hi = meta_ref[s, 3]
    rb = meta_ref[s, 1]
    rb_prev = meta_ref[jnp.maximum(s - 1, 0), 1]
    first = jnp.logical_or(s == 0, rb != rb_prev)

    hb = h_ref[...]
    he = lax.dot_general(hb, wfc_ref[0], (((1,), (1,)), ((), ())),
                         preferred_element_type=jnp.float32)
    he = he + bfc_ref[0]
    he = _gelu(he)
    yb = lax.dot_general(he, wproj_ref[0], (((1,), (1,)), ((), ())),
                         preferred_element_type=jnp.float32)
    yb = yb + bproj_ref[0]

    rows = lax.broadcasted_iota(jnp.int32, (GB, 1), 0)
    mask = jnp.logical_and(rows >= lo, rows < hi)

    @pl.when(first)
    def _():
        out_ref[...] = jnp.where(mask, yb, 0.0)

    @pl.when(jnp.logical_not(first))
    def _():
        out_ref[...] = jnp.where(mask, yb, out_ref[...])


def _grouped_gemm(meta, h_sorted, w_fc, b_fc, w_proj, b_proj):
    grid_spec = pltpu.PrefetchScalarGridSpec(
        num_scalar_prefetch=1,
        grid=(S_STEPS,),
        in_specs=[
            pl.BlockSpec((GB, D), lambda s, m: (m[s, 1], 0)),
            pl.BlockSpec((1, FF, D), lambda s, m: (m[s, 0], 0, 0)),
            pl.BlockSpec((1, 1, FF), lambda s, m: (m[s, 0], 0, 0)),
            pl.BlockSpec((1, D, FF), lambda s, m: (m[s, 0], 0, 0)),
            pl.BlockSpec((1, 1, D), lambda s, m: (m[s, 0], 0, 0)),
        ],
        out_specs=pl.BlockSpec((GB, D), lambda s, m: (m[s, 1], 0)),
    )
    return pl.pallas_call(
        _gemm_body,
        grid_spec=grid_spec,
        out_shape=jax.ShapeDtypeStruct((TK, D), jnp.float32),
    )(meta, h_sorted, w_fc, b_fc.reshape(E, 1, FF), w_proj,
      b_proj.reshape(E, 1, D))


def _combine_body(y_hbm, sr_hbm, g_hbm, offp_hbm,
                  out_hbm,
                  sr_v, off_v, pos0_v, pos1_v, g_v,
                  buf0, buf1, obuf, semg, semo):
    cid = lax.axis_index("c")
    sid = lax.axis_index("s")
    wid = sid * NC + cid
    base = wid * TWD

    pltpu.sync_copy(sr_hbm.at[pl.ds(base, TWD)], sr_v)
    pltpu.sync_copy(g_hbm.at[pl.ds(base, TWD)], g_v)
    pltpu.sync_copy(offp_hbm.at[0], off_v)

    _positions(sr_v, off_v, pos0_v, pos1_v)

    nch = TWD // CCH
    zero = jnp.zeros((LANES,), jnp.int32)

    def gather(ci):
        b = ci % 2
        c0 = pltpu.make_async_copy(y_hbm.at[pos0_v.at[ci]], buf0.at[b],
                                   semg.at[b])
        c1 = pltpu.make_async_copy(y_hbm.at[pos1_v.at[ci]], buf1.at[b],
                                   semg.at[b])
        return c0, c1

    def store(ci):
        return pltpu.make_async_copy(
            obuf, out_hbm.at[pl.ds(base + ci * CCH, CCH)], semo)

    g0c, g1c = gather(0)
    g0c.start()
    g1c.start()
    for ci in range(nch):
        b = ci % 2
        c0, c1 = gather(ci)
        c0.wait()
        c1.wait()
        if ci + 1 < nch:
            n0, n1 = gather(ci + 1)
            n0.start()
            n1.start()
        if ci >= 1:
            store(ci - 1).wait()

        def row_body(r, _):
            ridx = zero + (ci * CCH + r)
            g0v = plsc.load_gather(g_v, [ridx, zero])
            g1v = plsc.load_gather(g_v, [ridx, zero + 1])
            for j in range(D // LANES):
                a = buf0[b, r, pl.ds(j * LANES, LANES)]
                bb = buf1[b, r, pl.ds(j * LANES, LANES)]
                obuf[r, pl.ds(j * LANES, LANES)] = a * g0v + bb * g1v
            return 0

        lax.fori_loop(0, CCH, row_body, 0)
        store(ci).start()
    store(nch - 1).wait()


def _combine(y_sorted, selrank, gates, offp):
    mesh = plsc.VectorSubcoreMesh(core_axis_name="c", subcore_axis_name="s")
    kern = pl.kernel(
        _combine_body,
        out_type=jax.ShapeDtypeStruct((T, D), jnp.float32),
        mesh=mesh,
        compiler_params=pltpu.CompilerParams(needs_layout_passes=False),
        scratch_types=[
            pltpu.VMEM((TWD, 4), jnp.int32),
            pltpu.VMEM((16,), jnp.int32),
            pltpu.VMEM((TWD // CCH, CCH), jnp.int32),
            pltpu.VMEM((TWD // CCH, CCH), jnp.int32),
            pltpu.VMEM((TWD, 2), jnp.float32),
            pltpu.VMEM((2, CCH, D), jnp.float32),
            pltpu.VMEM((2, CCH, D), jnp.float32),
            pltpu.VMEM((CCH, D), jnp.float32),
            pltpu.SemaphoreType.DMA((2,)),
            pltpu.SemaphoreType.DMA,
        ],
    )
    return kern(y_sorted, selrank, gates, offp)


def kernel(hidden_states, gate_w, w_fc, b_fc, w_proj, b_proj):
    orig_shape = hidden_states.shape
    x = hidden_states.reshape(-1, D)

    logits, selrank, gates, meta, offp = _route(x, gate_w)
    h_sorted = _dispatch(x, selrank, offp)
    y_sorted = _grouped_gemm(meta, h_sorted, w_fc, b_fc, w_proj, b_proj)
    out = _combine(y_sorted, selrank, gates, offp)

    return (out.reshape(orig_shape), logits)


# speedup vs baseline: 1.1621x; 1.0007x over previous
import jax
import jax.numpy as jnp
from jax import lax
from jax.experimental import pallas as pl
from jax.experimental.pallas import tpu as pltpu
from jax.experimental.pallas import tpu_sc as plsc

E = 8
K = 2
D = 1024
FF = 2048
T = 4096
TK = T * K

NC = 2
NS = 16
NW = NC * NS
LANES = 16

TB = 512
GB = 512
NBLK = TK // GB
S_STEPS = NBLK + E - 1

TWD = T // NW
CCH = 16


def _route_body(x_ref, gw_ref, logits_ref, selrank_ref, gates_ref, meta_ref,
                offp_ref, carry_ref, carrylt_ref):
    step = pl.program_id(0)
    nsteps = pl.num_programs(0)

    @pl.when(step == 0)
    def _():
        carry_ref[...] = jnp.zeros_like(carry_ref)
        carrylt_ref[...] = jnp.zeros_like(carrylt_ref)

    xb = x_ref[...]
    gw = gw_ref[...]
    logits = lax.dot_general(xb, gw, (((1,), (1,)), ((), ())),
                             preferred_element_type=jnp.float32)
    logits_ref[...] = logits

    j8 = lax.broadcasted_iota(jnp.int32, (TB, E), 1)
    m0 = jnp.max(logits, axis=1, keepdims=True)
    is0 = logits == m0
    idx0 = jnp.min(jnp.where(is0, j8, E), axis=1, keepdims=True)
    masked = jnp.where(j8 == idx0, -jnp.inf, logits)
    m1 = jnp.max(masked, axis=1, keepdims=True)
    is1 = masked == m1
    idx1 = jnp.min(jnp.where(is1, j8, E), axis=1, keepdims=True)

    ex = jnp.exp(m1 - m0)
    denom = 1.0 + ex
    g0 = 1.0 / denom
    g1 = ex / denom

    oh0 = (j8 == idx0).astype(jnp.float32)
    oh1 = (j8 == idx1).astype(jnp.float32)
    oh = oh0 + oh1

    r_ids = lax.broadcasted_iota(jnp.int32, (TB, TB), 0)
    c_ids = lax.broadcasted_iota(jnp.int32, (TB, TB), 1)
    ltri = (c_ids < r_ids).astype(jnp.float32)
    cum_excl = carry_ref[...] + lax.dot_general(
        ltri, oh, (((1,), (0,)), ((), ())),
        preferred_element_type=jnp.float32)

    rank0 = jnp.sum(oh0 * cum_excl, axis=1, keepdims=True)
    rank1 = jnp.sum(oh1 * cum_excl, axis=1, keepdims=True)

    new_carry = carry_ref[...] + jnp.sum(oh, axis=0, keepdims=True)
    carry_ref[...] = new_carry

    ohlt = (idx0 < j8).astype(jnp.float32) + (idx1 < j8).astype(jnp.float32)
    new_carrylt = carrylt_ref[...] + jnp.sum(ohlt, axis=0, keepdims=True)
    carrylt_ref[...] = new_carrylt

    selrank_ref[...] = jnp.concatenate(
        [idx0, idx1, rank0.astype(jnp.int32), rank1.astype(jnp.int32)],
        axis=1)
    gates_ref[...] = jnp.concatenate([g0, g1], axis=1)

    @pl.when(step == nsteps - 1)
    def _():
        counts_f = new_carry
        ka = lax.broadcasted_iota(jnp.int32, (E, E), 0)
        eb = lax.broadcasted_iota(jnp.int32, (E, E), 1)
        mtri = (ka < eb).astype(jnp.float32)
        counts_i = counts_f.astype(jnp.int32)
        off_i = new_carrylt.astype(jnp.int32)
        off1_i = off_i + counts_i
        bstart = off_i // GB
        bend = (off1_i - 1) // GB
        nst = jnp.where(counts_i > 0, bend - bstart + 1, 0)
        first_f = lax.dot_general(nst.astype(jnp.float32), mtri,
                                  (((1,), (0,)), ((), ())),
                                  preferred_element_type=jnp.float32)
        first = first_f.astype(jnp.int32)
        cum_end = first + nst
        total = cum_end[:, E - 1:E]

        s2 = lax.broadcasted_iota(jnp.int32, (S_STEPS, E), 0)
        e2 = lax.broadcasted_iota(jnp.int32, (S_STEPS, E), 1)
        ge = (s2 >= cum_end).astype(jnp.int32)
        e_of = jnp.sum(ge, axis=1, keepdims=True)
        e_s = jnp.clip(e_of, 0, E - 1)
        onehot = (e2 == e_s).astype(jnp.float32)

        def pick(v_i):
            return jnp.sum(onehot * v_i.astype(jnp.float32), axis=1,
                           keepdims=True).astype(jnp.int32)

        s_col = lax.broadcasted_iota(jnp.int32, (S_STEPS, 1), 0)
        b_s = pick(bstart) + s_col - pick(first)
        lo = jnp.maximum(pick(off_i) - b_s * GB, 0)
        hi = jnp.minimum(pick(off1_i) - b_s * GB, GB)
        valid = s_col < total
        at_last = (s_col == total - 1)
        e_last = jnp.sum(jnp.where(at_last, e_s, 0), axis=0, keepdims=True)
        b_last = jnp.sum(jnp.where(at_last, b_s, 0), axis=0, keepdims=True)
        meta_ref[...] = jnp.concatenate(
            [jnp.where(valid, e_s, e_last),
             jnp.where(valid, b_s, b_last),
             jnp.where(valid, lo, 0),
             jnp.where(valid, hi, 0)], axis=1)
        offp_ref[...] = jnp.concatenate(
            [off_i, jnp.zeros((1, 16 - E), jnp.int32)], axis=1)


def _route(x, gate_w):
    nsteps = T // TB
    return pl.pallas_call(
        _route_body,
        grid=(nsteps,),
        in_specs=[
            pl.BlockSpec((TB, D), lambda s: (s, 0)),
            pl.BlockSpec((E, D), lambda s: (0, 0)),
        ],
        out_specs=[
            pl.BlockSpec((TB, E), lambda s: (s, 0)),
            pl.BlockSpec((TB, 4), lambda s: (s, 0)),
            pl.BlockSpec((TB, 2), lambda s: (s, 0)),
            pl.BlockSpec((S_STEPS, 4), lambda s: (0, 0)),
            pl.BlockSpec((1, 16), lambda s: (0, 0)),
        ],
        out_shape=[
            jax.ShapeDtypeStruct((T, E), jnp.float32),
            jax.ShapeDtypeStruct((T, 4), jnp.int32),
            jax.ShapeDtypeStruct((T, 2), jnp.float32),
            jax.ShapeDtypeStruct((S_STEPS, 4), jnp.int32),
            jax.ShapeDtypeStruct((1, 16), jnp.int32),
        ],
        scratch_shapes=[pltpu.VMEM((1, E), jnp.float32),
                        pltpu.VMEM((1, E), jnp.float32)],
    )(x, gate_w)


def _positions(sr_v, off_v, pos0_v, pos1_v):
    lane = lax.iota(jnp.int32, LANES)
    zero = jnp.zeros((LANES,), jnp.int32)
    for i in range(TWD // LANES):
        ti = lane + i * LANES
        s0 = plsc.load_gather(sr_v, [ti, zero])
        s1 = plsc.load_gather(sr_v, [ti, zero + 1])
        r0 = plsc.load_gather(sr_v, [ti, zero + 2])
        r1 = plsc.load_gather(sr_v, [ti, zero + 3])
        row = (i * LANES) // CCH
        col = (i * LANES) % CCH
        pos0_v[row, pl.ds(col, LANES)] = r0 + plsc.load_gather(off_v, [s0])
        pos1_v[row, pl.ds(col, LANES)] = r1 + plsc.load_gather(off_v, [s1])


def _dispatch_body(x_hbm, sr_hbm, offp_hbm, hs_hbm,
                   sr_v, off_v, pos0_v, pos1_v, xbuf, seml, sems):
    cid = lax.axis_index("c")
    sid = lax.axis_index("s")
    wid = sid * NC + cid
    base = wid * TWD

    pltpu.sync_copy(sr_hbm.at[pl.ds(base, TWD)], sr_v)
    pltpu.sync_copy(offp_hbm.at[0], off_v)

    _positions(sr_v, off_v, pos0_v, pos1_v)

    nch = TWD // CCH

    def load(ci):
        b = ci % 2
        return pltpu.make_async_copy(
            x_hbm.at[pl.ds(base + ci * CCH, CCH)], xbuf.at[b], seml.at[b])

    def scat(ci, slot):
        b = ci % 2
        pos = pos0_v if slot == 0 else pos1_v
        return pltpu.make_async_copy(
            xbuf.at[b], hs_hbm.at[pos.at[ci]], sems.at[b])

    load(0).start()
    for ci in range(nch):
        load(ci).wait()
        if ci + 1 < nch:
            if ci >= 1:
                scat(ci - 1, 0).wait()
                scat(ci - 1, 1).wait()
            load(ci + 1).start()
        scat(ci, 0).start()
        scat(ci, 1).start()
    if nch >= 2:
        scat(nch - 2, 0).wait()
        scat(nch - 2, 1).wait()
    scat(nch - 1, 0).wait()
    scat(nch - 1, 1).wait()


def _dispatch(x, selrank, offp):
    mesh = plsc.VectorSubcoreMesh(core_axis_name="c", subcore_axis_name="s")
    kern = pl.kernel(
        _dispatch_body,
        out_type=jax.ShapeDtypeStruct((TK, D), jnp.float32),
        mesh=mesh,
        compiler_params=pltpu.CompilerParams(needs_layout_passes=False),
        scratch_types=[
            pltpu.VMEM((TWD, 4), jnp.int32),
            pltpu.VMEM((16,), jnp.int32),
            pltpu.VMEM((TWD // CCH, CCH), jnp.int32),
            pltpu.VMEM((TWD // CCH, CCH), jnp.int32),
            pltpu.VMEM((2, CCH, D), jnp.float32),
            pltpu.SemaphoreType.DMA((2,)),
            pltpu.SemaphoreType.DMA((2,)),
        ],
    )
    return kern(x, selrank, offp)


def _gelu(x):
    return 0.5 * x * (1.0 + lax.erf(x * 0.7071067811865476))


def _gemm_body(meta_ref, h_ref, wfc_ref, bfc_ref, wproj_ref, bproj_ref,
               out_ref):
    s = pl.program_id(0)
    lo = meta_ref[s, 2]
    hi = meta_ref[s, 3]
    rb = meta_ref[s, 1]
    rb_prev = meta_ref[jnp.maximum(s - 1, 0), 1]
    first = jnp.logical_or(s == 0, rb != rb_prev)

    hb = h_ref[...]
    he = lax.dot_general(hb, wfc_ref[0], (((1,), (1,)), ((), ())),
                         preferred_element_type=jnp.float32)
    he = he + bfc_ref[0]
    he = _gelu(he)
    yb = lax.dot_general(he, wproj_ref[0], (((1,), (1,)), ((), ())),
                         preferred_element_type=jnp.float32)
    yb = yb + bproj_ref[0]

    rows = lax.broadcasted_iota(jnp.int32, (GB, 1), 0)
    mask = jnp.logical_and(rows >= lo, rows < hi)

    @pl.when(first)
    def _():
        out_ref[...] = jnp.where(mask, yb, 0.0)

    @pl.when(jnp.logical_not(first))
    def _():
        out_ref[...] = jnp.where(mask, yb, out_ref[...])


def _grouped_gemm(meta, h_sorted, w_fc, b_fc, w_proj, b_proj):
    grid_spec = pltpu.PrefetchScalarGridSpec(
        num_scalar_prefetch=1,
        grid=(S_STEPS,),
        in_specs=[
            pl.BlockSpec((GB, D), lambda s, m: (m[s, 1], 0)),
            pl.BlockSpec((1, FF, D), lambda s, m: (m[s, 0], 0, 0)),
            pl.BlockSpec((1, 1, FF), lambda s, m: (m[s, 0], 0, 0)),
            pl.BlockSpec((1, D, FF), lambda s, m: (m[s, 0], 0, 0)),
            pl.BlockSpec((1, 1, D), lambda s, m: (m[s, 0], 0, 0)),
        ],
        out_specs=pl.BlockSpec((GB, D), lambda s, m: (m[s, 1], 0)),
    )
    return pl.pallas_call(
        _gemm_body,
        grid_spec=grid_spec,
        out_shape=jax.ShapeDtypeStruct((TK, D), jnp.float32),
    )(meta, h_sorted, w_fc, b_fc.reshape(E, 1, FF), w_proj,
      b_proj.reshape(E, 1, D))


def _combine_body(y_hbm, sr_hbm, g_hbm, offp_hbm,
                  out_hbm,
                  sr_v, off_v, pos0_v, pos1_v, g_v,
                  buf0, buf1, obuf, semg, semo):
    cid = lax.axis_index("c")
    sid = lax.axis_index("s")
    wid = sid * NC + cid
    base = wid * TWD

    pltpu.sync_copy(sr_hbm.at[pl.ds(base, TWD)], sr_v)
    pltpu.sync_copy(g_hbm.at[pl.ds(base, TWD)], g_v)
    pltpu.sync_copy(offp_hbm.at[0], off_v)

    _positions(sr_v, off_v, pos0_v, pos1_v)

    nch = TWD // CCH
    zero = jnp.zeros((LANES,), jnp.int32)

    def gather(ci):
        b = ci % 2
        c0 = pltpu.make_async_copy(y_hbm.at[pos0_v.at[ci]], buf0.at[b],
                                   semg.at[b])
        c1 = pltpu.make_async_copy(y_hbm.at[pos1_v.at[ci]], buf1.at[b],
                                   semg.at[b])
        return c0, c1

    def store(ci):
        return pltpu.make_async_copy(
            obuf, out_hbm.at[pl.ds(base + ci * CCH, CCH)], semo)

    g0c, g1c = gather(0)
    g0c.start()
    g1c.start()
    for ci in range(nch):
        b = ci % 2
        c0, c1 = gather(ci)
        c0.wait()
        c1.wait()
        if ci + 1 < nch:
            n0, n1 = gather(ci + 1)
            n0.start()
            n1.start()
        if ci >= 1:
            store(ci - 1).wait()

        def row_body(r, _):
            ridx = zero + (ci * CCH + r)
            g0v = plsc.load_gather(g_v, [ridx, zero])
            g1v = plsc.load_gather(g_v, [ridx, zero + 1])
            for j in range(D // LANES):
                a = buf0[b, r, pl.ds(j * LANES, LANES)]
                bb = buf1[b, r, pl.ds(j * LANES, LANES)]
                obuf[r, pl.ds(j * LANES, LANES)] = a * g0v + bb * g1v
            return 0

        lax.fori_loop(0, CCH, row_body, 0)
        store(ci).start()
    store(nch - 1).wait()


def _combine(y_sorted, selrank, gates, offp):
    mesh = plsc.VectorSubcoreMesh(core_axis_name="c", subcore_axis_name="s")
    kern = pl.kernel(
        _combine_body,
        out_type=jax.ShapeDtypeStruct((T, D), jnp.float32),
        mesh=mesh,
        compiler_params=pltpu.CompilerParams(needs_layout_passes=False),
        scratch_types=[
            pltpu.VMEM((TWD, 4), jnp.int32),
            pltpu.VMEM((16,), jnp.int32),
            pltpu.VMEM((TWD // CCH, CCH), jnp.int32),
            pltpu.VMEM((TWD // CCH, CCH), jnp.int32),
            pltpu.VMEM((TWD, 2), jnp.float32),
            pltpu.VMEM((2, CCH, D), jnp.float32),
            pltpu.VMEM((2, CCH, D), jnp.float32),
            pltpu.VMEM((CCH, D), jnp.float32),
            pltpu.SemaphoreType.DMA((2,)),
            pltpu.SemaphoreType.DMA,
        ],
    )
    return kern(y_sorted, selrank, gates, offp)


def kernel(hidden_states, gate_w, w_fc, b_fc, w_proj, b_proj):
    orig_shape = hidden_states.shape
    x = hidden_states.reshape(-1, D)

    logits, selrank, gates, meta, offp = _route(x, gate_w)
    h_sorted = _dispatch(x, selrank, offp)
    y_sorted = _grouped_gemm(meta, h_sorted, w_fc, b_fc, w_proj, b_proj)
    out = _combine(y_sorted, selrank, gates, offp)

    return (out.reshape(orig_shape), logits)
